# baseline (speedup 1.0000x reference)
import functools

import jax
import jax.numpy as jnp
from jax import lax
from jax.experimental import pallas as pl
from jax.experimental.pallas import tpu as pltpu
from jax.experimental.pallas import tpu_sc as plsc

EOS = 1e-10
NNODES = 10000
NEDGES = 160000
DIN = 256
EMB = 128
NCLASS = 16
NPRED = 80000
TAU = 0.1

NC = 2
NS = 16
NW = NC * NS
L = 16

NPADN = 10240
STRIPE = NPADN // NS

CH = 128
NCH_P = 20
PRED_PT = NCH_P * CH
PPAD = NW * PRED_PT
NCH_N = 44
NON_PT = NCH_N * CH
NONPAD = NW * NON_PT
NCH_A = NCH_P + NCH_N
BLK = 8
DCH = 64
DNCH = (PRED_PT + NON_PT) // DCH
DBLK = 4
NBLK = DNCH // DBLK
RING = 4

_f32 = jnp.float32
_i32 = jnp.int32

_SC_PARAMS = pltpu.CompilerParams(needs_layout_passes=False)


def _h_body(x_ref, wl_ref, bl_ref, h_ref):
    x = x_ref[...]
    h = jnp.dot(x, wl_ref[...], preferred_element_type=_f32) + bl_ref[...]
    h_ref[...] = jnp.maximum(h, 0.0)


def _emb_body(emb_ref, embn_ref):
    e = emb_ref[...]
    n2 = jnp.sum(e * e, axis=1, keepdims=True)
    embn_ref[...] = e * lax.rsqrt(jnp.maximum(n2, 1e-16))


def _rs_body(degt_ref, h_ref, h2_ref, rsi_ref):
    d = degt_ref[...]
    tot_o = d[:, 0:1] + d[:, 2:3]
    tot_i = d[:, 1:2] + d[:, 3:4]
    rso = lax.rsqrt(jnp.maximum(tot_o, 1e-12))
    rsi_ref[...] = lax.rsqrt(jnp.maximum(tot_i, 1e-12))
    h2_ref[...] = h_ref[...] * rso[:NNODES]


def _mlp_body(p_ref, rsi_ref, wm_ref, bm_ref, o_ref):
    hsum = (p_ref[0] + p_ref[1]) * rsi_ref[...]
    o_ref[...] = jnp.dot(hsum, wm_ref[...], preferred_element_type=_f32) + bm_ref[...]


def _edge_w_body(embn_hbm, i0_hbm, i1_hbm, srcn_hbm, dstn_hbm, wb_hbm,
                 wpred_hbm, deg_hbm,
                 i0_v, i1_v, a0_v, b0_v, a1_v, b1_v, wst_v, wnb_v,
                 srcn_v, dstn_v, zz_v, dego_sp, degi_sp,
                 sa0, sb0, sa1, sb1):
    c = lax.axis_index("c")
    s = lax.axis_index("s")
    wid = c * NS + s
    iota = lax.iota(_i32, L)

    def _z(i, _):
        zz_v[pl.ds(i * L, L)] = jnp.zeros((L,), _f32)
        return 0
    lax.fori_loop(0, STRIPE // L, _z, 0)
    pltpu.sync_copy(zz_v, dego_sp.at[pl.ds(s * STRIPE, STRIPE)])
    pltpu.sync_copy(zz_v, degi_sp.at[pl.ds(s * STRIPE, STRIPE)])
    plsc.subcore_barrier()

    pltpu.sync_copy(i0_hbm.at[wid], i0_v)
    pltpu.sync_copy(i1_hbm.at[wid], i1_v)
    pltpu.sync_copy(srcn_hbm.at[wid], srcn_v)
    pltpu.sync_copy(dstn_hbm.at[wid], dstn_v)
    pltpu.sync_copy(wb_hbm.at[wid], wnb_v)

    bufs = ((a0_v, b0_v, sa0, sb0), (a1_v, b1_v, sa1, sb1))

    pltpu.async_copy(embn_hbm.at[i0_v.at[0]], a0_v, sa0)
    pltpu.async_copy(embn_hbm.at[i1_v.at[0]], b0_v, sb0)

    def _pred_pair(t, _):
        for par in range(2):
            ci = t * 2 + par
            abuf, bbuf, sa, sb = bufs[par]
            oabuf, obbuf, osa, osb = bufs[1 - par]
            pltpu.make_async_copy(embn_hbm.at[i0_v.at[ci]], abuf, sa).wait()
            pltpu.make_async_copy(embn_hbm.at[i1_v.at[ci]], bbuf, sb).wait()

            @pl.when(ci + 1 < NCH_P)
            def _():
                pltpu.async_copy(embn_hbm.at[i0_v.at[ci + 1]], oabuf, osa)
                pltpu.async_copy(embn_hbm.at[i1_v.at[ci + 1]], obbuf, osb)

            for q in range(2):
                nj = ci * 2 + q
                pltpu.sync_copy(wnb_v.at[nj], dego_sp.at[srcn_v.at[nj]], add=True)
                pltpu.sync_copy(wnb_v.at[nj], degi_sp.at[dstn_v.at[nj]], add=True)

            base = wid * PRED_PT + ci * CH

            def _group(g, _):
                def _edge(j, wvec):
                    e = g * L + j
                    acc = jnp.zeros((L,), _f32)
                    for k in range(EMB // L):
                        acc = acc + abuf[e, pl.ds(k * L, L)] * bbuf[e, pl.ds(k * L, L)]
                    cos = jnp.sum(acc)
                    valid = (base + e) < NPRED
                    w_sc = jnp.where(valid & (cos >= TAU), cos + EOS, 0.0)
                    return jnp.where(iota == j, w_sc, wvec)
                w16 = lax.fori_loop(0, L, _edge, jnp.zeros((L,), _f32))
                wst_v[ci, pl.ds(g * L, L)] = w16
                return 0
            lax.fori_loop(0, CH // L, _group, 0)

            pltpu.sync_copy(wst_v.at[ci], dego_sp.at[i0_v.at[ci]], add=True)
            pltpu.sync_copy(wst_v.at[ci], degi_sp.at[i1_v.at[ci]], add=True)
        return 0
    lax.fori_loop(0, NCH_P // 2, _pred_pair, 0)
    pltpu.sync_copy(wst_v, wpred_hbm.at[wid])

    def _non_chunk(ci, _):
        pltpu.sync_copy(wnb_v.at[ci], dego_sp.at[srcn_v.at[ci]], add=True)
        pltpu.sync_copy(wnb_v.at[ci], degi_sp.at[dstn_v.at[ci]], add=True)
        return 0
    lax.fori_loop(2 * NCH_P, NCH_N, _non_chunk, 0)

    plsc.subcore_barrier()
    pltpu.sync_copy(dego_sp.at[pl.ds(s * STRIPE, STRIPE)],
                    deg_hbm.at[c, 0, pl.ds(s * STRIPE, STRIPE)])
    pltpu.sync_copy(degi_sp.at[pl.ds(s * STRIPE, STRIPE)],
                    deg_hbm.at[c, 1, pl.ds(s * STRIPE, STRIPE)])


def _prop_body(h_hbm, src_hbm, dst_hbm, w_hbm,
               p_hbm,
               sb_v, db_v, wb_v, rows0, rows1, rows2, rows3, acc_sp,
               g0, g1, g2, g3, s0, s1, s2, s3):
    c = lax.axis_index("c")
    s = lax.axis_index("s")
    wid = c * NS + s

    def _zr(r, _):
        for k in range(EMB // L):
            rows0[r, pl.ds(k * L, L)] = jnp.zeros((L,), _f32)
        return 0
    lax.fori_loop(0, DCH, _zr, 0)
    for j in range(STRIPE // DCH):
        pltpu.async_copy(rows0, acc_sp.at[pl.ds(s * STRIPE + j * DCH, DCH)], g1)
    for j in range(STRIPE // DCH):
        pltpu.make_async_copy(rows0, acc_sp.at[pl.ds(s * STRIPE, DCH)], g1).wait()
    plsc.subcore_barrier()

    pltpu.sync_copy(src_hbm.at[wid, pl.ds(0, DBLK)], sb_v.at[0])
    pltpu.sync_copy(src_hbm.at[wid, pl.ds(DBLK, DBLK)], sb_v.at[1])

    bufs = ((rows0, g0, s0), (rows1, g1, s1), (rows2, g2, s2), (rows3, g3, s3))
    for r in range(RING - 1):
        pltpu.async_copy(h_hbm.at[sb_v.at[0, r]], bufs[r][0], bufs[r][1])

    def _block(bi, _):
        pltpu.sync_copy(dst_hbm.at[wid, pl.ds(bi * DBLK, DBLK)], db_v)
        pltpu.sync_copy(w_hbm.at[wid, pl.ds(bi * DBLK, DBLK)], wb_v)
        bslot = lax.rem(bi, 2)

        for par in range(RING):
            lc = par
            ci = bi * DBLK + lc
            buf, gsem, ssem = bufs[par]
            q = (par + RING - 1) % RING
            qbuf, qgsem, qssem = bufs[q]

            pltpu.make_async_copy(h_hbm.at[sb_v.at[bslot, lc]], buf,
                                  gsem).wait()

            @pl.when(ci > 0)
            def _():
                pltpu.make_async_copy(qbuf, acc_sp.at[db_v.at[lc]],
                                      qssem).wait()

            @pl.when(ci + RING - 1 < DNCH)
            def _():
                tj = ci + RING - 1
                tslot = lax.rem(lax.div(tj, DBLK), 2)
                trow = lax.rem(tj, DBLK)
                pltpu.async_copy(h_hbm.at[sb_v.at[tslot, trow]], qbuf, qgsem)

            def _sg(g, _):
                w16 = wb_v[lc, pl.ds(g * L, L)]
                for j in range(L):
                    b = w16[j]
                    e = g * L + j
                    for k in range(EMB // L):
                        buf[e, pl.ds(k * L, L)] = buf[e, pl.ds(k * L, L)] * b
                return 0
            lax.fori_loop(0, DCH // L, _sg, 0)

            pltpu.async_copy(buf, acc_sp.at[db_v.at[lc]], ssem, add=True)

        @pl.when(bi + 2 < NBLK)
        def _():
            pltpu.sync_copy(src_hbm.at[wid, pl.ds((bi + 2) * DBLK, DBLK)],
                            sb_v.at[bslot])
        return 0
    lax.fori_loop(0, NBLK, _block, 0)

    pltpu.make_async_copy(rows3, acc_sp.at[db_v.at[DBLK - 1]], s3).wait()

    plsc.subcore_barrier()
    for j in range(STRIPE // CH):
        sl = pl.ds(s * STRIPE + j * CH, CH)
        pltpu.async_copy(acc_sp.at[sl], p_hbm.at[c, sl], g0)
    for j in range(STRIPE // CH):
        pltpu.make_async_copy(acc_sp.at[pl.ds(s * STRIPE, CH)],
                              p_hbm.at[c, pl.ds(s * STRIPE, CH)], g0).wait()


def kernel(features, embedding, weights_lp, W_lin, b_lin, W_mlp, b_mlp,
           edges, pred_edge_index):
    edges = edges.astype(_i32)
    pei = pred_edge_index.astype(_i32)

    npp = PPAD - NPRED
    i0p = jnp.concatenate([pei[0], jnp.arange(npp, dtype=_i32) % NNODES])
    i1p = jnp.concatenate([pei[1], (jnp.arange(npp, dtype=_i32) + 7) % NNODES])
    nnp = NONPAD - (NEDGES + NNODES)
    sl = jnp.arange(NNODES, dtype=_i32)
    srcn = jnp.concatenate([edges[0], sl, jnp.arange(nnp, dtype=_i32) % NNODES])
    dstn = jnp.concatenate([edges[1], sl, (jnp.arange(nnp, dtype=_i32) + 13) % NNODES])
    w_base = jnp.concatenate([
        weights_lp + EOS,
        jnp.ones((NNODES,), _f32) + EOS,
        jnp.zeros((nnp,), _f32),
    ])
    i0p3 = i0p.reshape(NW, NCH_P, CH)
    i1p3 = i1p.reshape(NW, NCH_P, CH)
    srcn3 = srcn.reshape(NW, NCH_N, CH)
    dstn3 = dstn.reshape(NW, NCH_N, CH)
    wb3 = w_base.reshape(NW, NCH_N, CH)

    emb_n = pl.pallas_call(
        _emb_body,
        out_shape=jax.ShapeDtypeStruct((NNODES, EMB), _f32),
    )(embedding)
    h = pl.pallas_call(
        _h_body,
        out_shape=jax.ShapeDtypeStruct((NNODES, EMB), _f32),
    )(features, W_lin.T, b_lin.reshape(1, EMB))

    sc_mesh = plsc.VectorSubcoreMesh(core_axis_name="c", subcore_axis_name="s")
    edge_w = pl.kernel(
        _edge_w_body,
        out_type=[jax.ShapeDtypeStruct((NW, NCH_P, CH), _f32),
                  jax.ShapeDtypeStruct((NC, 2, NPADN), _f32)],
        mesh=sc_mesh,
        compiler_params=_SC_PARAMS,
        scratch_types=[
            pltpu.VMEM((NCH_P, CH), _i32),
            pltpu.VMEM((NCH_P, CH), _i32),
            pltpu.VMEM((CH, EMB), _f32),
            pltpu.VMEM((CH, EMB), _f32),
            pltpu.VMEM((CH, EMB), _f32),
            pltpu.VMEM((CH, EMB), _f32),
            pltpu.VMEM((NCH_P, CH), _f32),
            pltpu.VMEM((NCH_N, CH), _f32),
            pltpu.VMEM((NCH_N, CH), _i32),
            pltpu.VMEM((NCH_N, CH), _i32),
            pltpu.VMEM((STRIPE,), _f32),
            pltpu.VMEM_SHARED((NPADN,), _f32),
            pltpu.VMEM_SHARED((NPADN,), _f32),
            pltpu.SemaphoreType.DMA,
            pltpu.SemaphoreType.DMA,
            pltpu.SemaphoreType.DMA,
            pltpu.SemaphoreType.DMA,
        ],
    )
    wpred3, deg = edge_w(emb_n, i0p3, i1p3, srcn3, dstn3, wb3)

    deg_t = deg.reshape(NC * 2, NPADN).T
    h2, rsi_col = pl.pallas_call(
        _rs_body,
        out_shape=[jax.ShapeDtypeStruct((NNODES, EMB), _f32),
                   jax.ShapeDtypeStruct((NPADN, 1), _f32)],
    )(deg_t, h)

    src_all = jnp.concatenate([i0p, srcn]).reshape(NW, DNCH, DCH)
    dst_all = jnp.concatenate([i1p, dstn]).reshape(NW, DNCH, DCH)
    w_all = jnp.concatenate([wpred3.reshape(PPAD), w_base]).reshape(NW, DNCH, DCH)

    prop = pl.kernel(
        _prop_body,
        out_type=jax.ShapeDtypeStruct((NC, NPADN, EMB), _f32),
        mesh=sc_mesh,
        compiler_params=_SC_PARAMS,
        scratch_types=[
            pltpu.VMEM((2, DBLK, DCH), _i32),
            pltpu.VMEM((DBLK, DCH), _i32),
            pltpu.VMEM((DBLK, DCH), _f32),
            pltpu.VMEM((DCH, EMB), _f32),
            pltpu.VMEM((DCH, EMB), _f32),
            pltpu.VMEM((DCH, EMB), _f32),
            pltpu.VMEM((DCH, EMB), _f32),
            pltpu.VMEM_SHARED((NPADN, EMB), _f32),
        ] + [pltpu.SemaphoreType.DMA] * 8,
    )
    p = prop(h2, src_all, dst_all, w_all)

    wm_pad = jnp.zeros((EMB, EMB), _f32).at[:, :NCLASS].set(W_mlp.T)
    bm_pad = jnp.zeros((1, EMB), _f32).at[0, :NCLASS].set(b_mlp)
    out_pad = pl.pallas_call(
        _mlp_body,
        out_shape=jax.ShapeDtypeStruct((NPADN, EMB), _f32),
    )(p, rsi_col, wm_pad, bm_pad)
    return out_pad[:NNODES, :NCLASS]


# speedup vs baseline: 1.1068x; 1.1068x over previous
import functools

import jax
import jax.numpy as jnp
from jax import lax
from jax.experimental import pallas as pl
from jax.experimental.pallas import tpu as pltpu
from jax.experimental.pallas import tpu_sc as plsc

EOS = 1e-10
NNODES = 10000
NEDGES = 160000
DIN = 256
EMB = 128
NCLASS = 16
NPRED = 80000
TAU = 0.1

NC = 2
NS = 16
NW = NC * NS
L = 16

NPADN = 10240
STRIPE = NPADN // NS

CH = 128
NCH_P = 20
PRED_PT = NCH_P * CH
PPAD = NW * PRED_PT
NCH_N = 44
NON_PT = NCH_N * CH
NONPAD = NW * NON_PT
NCH_A = NCH_P + NCH_N
BLK = 8

_f32 = jnp.float32
_i32 = jnp.int32

_SC_PARAMS = pltpu.CompilerParams(needs_layout_passes=False)


def _h_body(x_ref, wl_ref, bl_ref, h_ref):
    x = x_ref[...]
    h = jnp.dot(x, wl_ref[...], preferred_element_type=_f32) + bl_ref[...]
    h_ref[...] = jnp.maximum(h, 0.0)


def _emb_body(emb_ref, embn_ref):
    e = emb_ref[...]
    n2 = jnp.sum(e * e, axis=1, keepdims=True)
    embn_ref[...] = e * lax.rsqrt(jnp.maximum(n2, 1e-16))


def _rs_body(degt_ref, h_ref, h2_ref, rsi_ref):
    d = degt_ref[...]
    tot_o = d[:, 0:1] + d[:, 2:3]
    tot_i = d[:, 1:2] + d[:, 3:4]
    rso = lax.rsqrt(jnp.maximum(tot_o, 1e-12))
    rsi_ref[...] = lax.rsqrt(jnp.maximum(tot_i, 1e-12))
    h2_ref[...] = h_ref[...] * rso[:NNODES]


def _mlp_body(p_ref, rsi_ref, wm_ref, bm_ref, o_ref):
    hsum = (p_ref[0] + p_ref[1]) * rsi_ref[...]
    o_ref[...] = jnp.dot(hsum, wm_ref[...], preferred_element_type=_f32) + bm_ref[...]


def _edge_w_body(embn_hbm, i0_hbm, i1_hbm, srcn_hbm, dstn_hbm, wb_hbm,
                 wpred_hbm, deg_hbm,
                 i0_v, i1_v, a0_v, b0_v, a1_v, b1_v, wst_v, wnb_v,
                 srcn_v, dstn_v, zz_v, dego_sp, degi_sp,
                 sa0, sb0, sa1, sb1):
    c = lax.axis_index("c")
    s = lax.axis_index("s")
    wid = c * NS + s
    iota = lax.iota(_i32, L)

    def _z(i, _):
        zz_v[pl.ds(i * L, L)] = jnp.zeros((L,), _f32)
        return 0
    lax.fori_loop(0, STRIPE // L, _z, 0)
    pltpu.sync_copy(zz_v, dego_sp.at[pl.ds(s * STRIPE, STRIPE)])
    pltpu.sync_copy(zz_v, degi_sp.at[pl.ds(s * STRIPE, STRIPE)])
    plsc.subcore_barrier()

    pltpu.sync_copy(i0_hbm.at[wid], i0_v)
    pltpu.sync_copy(i1_hbm.at[wid], i1_v)
    pltpu.sync_copy(srcn_hbm.at[wid], srcn_v)
    pltpu.sync_copy(dstn_hbm.at[wid], dstn_v)
    pltpu.sync_copy(wb_hbm.at[wid], wnb_v)

    bufs = ((a0_v, b0_v, sa0, sb0), (a1_v, b1_v, sa1, sb1))

    pltpu.async_copy(embn_hbm.at[i0_v.at[0]], a0_v, sa0)
    pltpu.async_copy(embn_hbm.at[i1_v.at[0]], b0_v, sb0)

    def _pred_pair(t, _):
        for par in range(2):
            ci = t * 2 + par
            abuf, bbuf, sa, sb = bufs[par]
            oabuf, obbuf, osa, osb = bufs[1 - par]
            pltpu.make_async_copy(embn_hbm.at[i0_v.at[ci]], abuf, sa).wait()
            pltpu.make_async_copy(embn_hbm.at[i1_v.at[ci]], bbuf, sb).wait()

            @pl.when(ci + 1 < NCH_P)
            def _():
                pltpu.async_copy(embn_hbm.at[i0_v.at[ci + 1]], oabuf, osa)
                pltpu.async_copy(embn_hbm.at[i1_v.at[ci + 1]], obbuf, osb)

            for q in range(2):
                nj = ci * 2 + q
                pltpu.sync_copy(wnb_v.at[nj], dego_sp.at[srcn_v.at[nj]], add=True)
                pltpu.sync_copy(wnb_v.at[nj], degi_sp.at[dstn_v.at[nj]], add=True)

            base = wid * PRED_PT + ci * CH

            def _group(g, _):
                def _edge(j, wvec):
                    e = g * L + j
                    acc = jnp.zeros((L,), _f32)
                    for k in range(EMB // L):
                        acc = acc + abuf[e, pl.ds(k * L, L)] * bbuf[e, pl.ds(k * L, L)]
                    cos = jnp.sum(acc)
                    valid = (base + e) < NPRED
                    w_sc = jnp.where(valid & (cos >= TAU), cos + EOS, 0.0)
                    return jnp.where(iota == j, w_sc, wvec)
                w16 = lax.fori_loop(0, L, _edge, jnp.zeros((L,), _f32))
                wst_v[ci, pl.ds(g * L, L)] = w16
                return 0
            lax.fori_loop(0, CH // L, _group, 0)

            pltpu.sync_copy(wst_v.at[ci], dego_sp.at[i0_v.at[ci]], add=True)
            pltpu.sync_copy(wst_v.at[ci], degi_sp.at[i1_v.at[ci]], add=True)
        return 0
    lax.fori_loop(0, NCH_P // 2, _pred_pair, 0)
    pltpu.sync_copy(wst_v, wpred_hbm.at[wid])

    def _non_chunk(ci, _):
        pltpu.sync_copy(wnb_v.at[ci], dego_sp.at[srcn_v.at[ci]], add=True)
        pltpu.sync_copy(wnb_v.at[ci], degi_sp.at[dstn_v.at[ci]], add=True)
        return 0
    lax.fori_loop(2 * NCH_P, NCH_N, _non_chunk, 0)

    plsc.subcore_barrier()
    pltpu.sync_copy(dego_sp.at[pl.ds(s * STRIPE, STRIPE)],
                    deg_hbm.at[c, 0, pl.ds(s * STRIPE, STRIPE)])
    pltpu.sync_copy(degi_sp.at[pl.ds(s * STRIPE, STRIPE)],
                    deg_hbm.at[c, 1, pl.ds(s * STRIPE, STRIPE)])


def _prop_body(h_hbm, src_hbm, dst_hbm, w_hbm,
               p_hbm,
               sb_v, db_v, wb_v, rows0, rows1, acc_sp, g0, g1, s0, s1):
    c = lax.axis_index("c")
    s = lax.axis_index("s")
    wid = c * NS + s

    def _zr(r, _):
        for k in range(EMB // L):
            rows0[r, pl.ds(k * L, L)] = jnp.zeros((L,), _f32)
        return 0
    lax.fori_loop(0, CH, _zr, 0)
    for j in range(STRIPE // CH):
        pltpu.async_copy(rows0, acc_sp.at[pl.ds(s * STRIPE + j * CH, CH)], g1)
    for j in range(STRIPE // CH):
        pltpu.make_async_copy(rows0, acc_sp.at[pl.ds(s * STRIPE, CH)], g1).wait()
    plsc.subcore_barrier()

    pltpu.sync_copy(src_hbm.at[wid], sb_v)
    pltpu.async_copy(h_hbm.at[sb_v.at[0]], rows0, g0)

    bufs = ((rows0, g0, s0), (rows1, g1, s1))

    def _block(bi, _):
        pltpu.sync_copy(dst_hbm.at[wid, pl.ds(bi * BLK, BLK)], db_v)
        pltpu.sync_copy(w_hbm.at[wid, pl.ds(bi * BLK, BLK)], wb_v)

        def _pair(t, _):
            for par in range(2):
                lc = t * 2 + par
                ci = bi * BLK + lc
                buf, gsem, ssem = bufs[par]
                obuf, ogsem, ossem = bufs[1 - par]

                pltpu.make_async_copy(h_hbm.at[sb_v.at[ci]], buf, gsem).wait()

                @pl.when(ci > 0)
                def _():
                    pltpu.make_async_copy(obuf, acc_sp.at[db_v.at[lc]], ossem).wait()

                @pl.when(ci + 1 < NCH_A)
                def _():
                    pltpu.async_copy(h_hbm.at[sb_v.at[ci + 1]], obuf, ogsem)

                def _sg(g, _):
                    w16 = wb_v[lc, pl.ds(g * L, L)]
                    for j in range(L):
                        b = w16[j]
                        e = g * L + j
                        for k in range(EMB // L):
                            buf[e, pl.ds(k * L, L)] = buf[e, pl.ds(k * L, L)] * b
                    return 0
                lax.fori_loop(0, CH // L, _sg, 0)

                pltpu.async_copy(buf, acc_sp.at[db_v.at[lc]], ssem, add=True)
            return 0
        lax.fori_loop(0, BLK // 2, _pair, 0)
        return 0
    lax.fori_loop(0, NCH_A // BLK, _block, 0)

    pltpu.make_async_copy(rows1, acc_sp.at[db_v.at[BLK - 1]], s1).wait()

    plsc.subcore_barrier()
    for j in range(STRIPE // CH):
        sl = pl.ds(s * STRIPE + j * CH, CH)
        pltpu.async_copy(acc_sp.at[sl], p_hbm.at[c, sl], g0)
    for j in range(STRIPE // CH):
        pltpu.make_async_copy(acc_sp.at[pl.ds(s * STRIPE, CH)],
                              p_hbm.at[c, pl.ds(s * STRIPE, CH)], g0).wait()


def kernel(features, embedding, weights_lp, W_lin, b_lin, W_mlp, b_mlp,
           edges, pred_edge_index):
    edges = edges.astype(_i32)
    pei = pred_edge_index.astype(_i32)

    npp = PPAD - NPRED
    i0p = jnp.concatenate([pei[0], jnp.arange(npp, dtype=_i32) % NNODES])
    i1p = jnp.concatenate([pei[1], (jnp.arange(npp, dtype=_i32) + 7) % NNODES])
    nnp = NONPAD - (NEDGES + NNODES)
    sl = jnp.arange(NNODES, dtype=_i32)
    srcn = jnp.concatenate([edges[0], sl, jnp.arange(nnp, dtype=_i32) % NNODES])
    dstn = jnp.concatenate([edges[1], sl, (jnp.arange(nnp, dtype=_i32) + 13) % NNODES])
    w_base = jnp.concatenate([
        weights_lp + EOS,
        jnp.ones((NNODES,), _f32) + EOS,
        jnp.zeros((nnp,), _f32),
    ])
    i0p3 = i0p.reshape(NW, NCH_P, CH)
    i1p3 = i1p.reshape(NW, NCH_P, CH)
    srcn3 = srcn.reshape(NW, NCH_N, CH)
    dstn3 = dstn.reshape(NW, NCH_N, CH)
    wb3 = w_base.reshape(NW, NCH_N, CH)

    emb_n = pl.pallas_call(
        _emb_body,
        out_shape=jax.ShapeDtypeStruct((NNODES, EMB), _f32),
    )(embedding)
    h = pl.pallas_call(
        _h_body,
        out_shape=jax.ShapeDtypeStruct((NNODES, EMB), _f32),
    )(features, W_lin.T, b_lin.reshape(1, EMB))

    sc_mesh = plsc.VectorSubcoreMesh(core_axis_name="c", subcore_axis_name="s")
    edge_w = pl.kernel(
        _edge_w_body,
        out_type=[jax.ShapeDtypeStruct((NW, NCH_P, CH), _f32),
                  jax.ShapeDtypeStruct((NC, 2, NPADN), _f32)],
        mesh=sc_mesh,
        compiler_params=_SC_PARAMS,
        scratch_types=[
            pltpu.VMEM((NCH_P, CH), _i32),
            pltpu.VMEM((NCH_P, CH), _i32),
            pltpu.VMEM((CH, EMB), _f32),
            pltpu.VMEM((CH, EMB), _f32),
            pltpu.VMEM((CH, EMB), _f32),
            pltpu.VMEM((CH, EMB), _f32),
            pltpu.VMEM((NCH_P, CH), _f32),
            pltpu.VMEM((NCH_N, CH), _f32),
            pltpu.VMEM((NCH_N, CH), _i32),
            pltpu.VMEM((NCH_N, CH), _i32),
            pltpu.VMEM((STRIPE,), _f32),
            pltpu.VMEM_SHARED((NPADN,), _f32),
            pltpu.VMEM_SHARED((NPADN,), _f32),
            pltpu.SemaphoreType.DMA,
            pltpu.SemaphoreType.DMA,
            pltpu.SemaphoreType.DMA,
            pltpu.SemaphoreType.DMA,
        ],
    )
    wpred3, deg = edge_w(emb_n, i0p3, i1p3, srcn3, dstn3, wb3)

    deg_t = deg.reshape(NC * 2, NPADN).T
    h2, rsi_col = pl.pallas_call(
        _rs_body,
        out_shape=[jax.ShapeDtypeStruct((NNODES, EMB), _f32),
                   jax.ShapeDtypeStruct((NPADN, 1), _f32)],
    )(deg_t, h)

    src_all = jnp.concatenate([i0p, srcn]).reshape(NW, NCH_A, CH)
    dst_all = jnp.concatenate([i1p, dstn]).reshape(NW, NCH_A, CH)
    w_all = jnp.concatenate([wpred3.reshape(PPAD), w_base]).reshape(NW, NCH_A, CH)

    prop = pl.kernel(
        _prop_body,
        out_type=jax.ShapeDtypeStruct((NC, NPADN, EMB), _f32),
        mesh=sc_mesh,
        compiler_params=_SC_PARAMS,
        scratch_types=[
            pltpu.VMEM((NCH_A, CH), _i32),
            pltpu.VMEM((BLK, CH), _i32),
            pltpu.VMEM((BLK, CH), _f32),
            pltpu.VMEM((CH, EMB), _f32),
            pltpu.VMEM((CH, EMB), _f32),
            pltpu.VMEM_SHARED((NPADN, EMB), _f32),
            pltpu.SemaphoreType.DMA,
            pltpu.SemaphoreType.DMA,
            pltpu.SemaphoreType.DMA,
            pltpu.SemaphoreType.DMA,
        ],
    )
    p = prop(h2, src_all, dst_all, w_all)

    wm_pad = jnp.zeros((EMB, EMB), _f32).at[:, :NCLASS].set(W_mlp.T)
    bm_pad = jnp.zeros((1, EMB), _f32).at[0, :NCLASS].set(b_mlp)
    out_pad = pl.pallas_call(
        _mlp_body,
        out_shape=jax.ShapeDtypeStruct((NPADN, EMB), _f32),
    )(p, rsi_col, wm_pad, bm_pad)
    return out_pad[:NNODES, :NCLASS]


# speedup vs baseline: 1.1114x; 1.0042x over previous
import functools

import jax
import jax.numpy as jnp
from jax import lax
from jax.experimental import pallas as pl
from jax.experimental.pallas import tpu as pltpu
from jax.experimental.pallas import tpu_sc as plsc

EOS = 1e-10
NNODES = 10000
NEDGES = 160000
DIN = 256
EMB = 128
NCLASS = 16
NPRED = 80000
TAU = 0.1

NC = 2
NS = 16
NW = NC * NS
L = 16

NPADN = 10240
STRIPE = NPADN // NS

CH = 128
NCH_P = 20
PRED_PT = NCH_P * CH
PPAD = NW * PRED_PT
NCH_N = 44
NON_PT = NCH_N * CH
NONPAD = NW * NON_PT
NCH_A = NCH_P + NCH_N
BLK = 8

_f32 = jnp.float32
_i32 = jnp.int32

_SC_PARAMS = pltpu.CompilerParams(needs_layout_passes=False)


def _h_body(x_ref, wl_ref, bl_ref, h_ref):
    x = x_ref[...]
    h = jnp.dot(x, wl_ref[...], preferred_element_type=_f32) + bl_ref[...]
    h_ref[...] = jnp.maximum(h, 0.0)


def _emb_body(emb_ref, embn_ref):
    e = emb_ref[...]
    n2 = jnp.sum(e * e, axis=1, keepdims=True)
    embn_ref[...] = e * lax.rsqrt(jnp.maximum(n2, 1e-16))


def _rs_body(degt_ref, h_ref, h2_ref, rsi_ref):
    d = degt_ref[...]
    tot_o = d[:, 0:1] + d[:, 2:3]
    tot_i = d[:, 1:2] + d[:, 3:4]
    rso = lax.rsqrt(jnp.maximum(tot_o, 1e-12))
    rsi_ref[...] = lax.rsqrt(jnp.maximum(tot_i, 1e-12))
    h2_ref[...] = h_ref[...] * rso[:NNODES]


def _mlp_body(p_ref, rsi_ref, wm_ref, bm_ref, o_ref):
    hsum = (p_ref[0] + p_ref[1]) * rsi_ref[...]
    o_ref[...] = jnp.dot(hsum, wm_ref[...], preferred_element_type=_f32) + bm_ref[...]


def _edge_w_body(embn_hbm, i0_hbm, i1_hbm, srcn_hbm, dstn_hbm, wb_hbm,
                 wpred_hbm, deg_hbm,
                 i0_v, i1_v, a0_v, b0_v, a1_v, b1_v, wst_v, wnb_v,
                 srcn_v, dstn_v, zz_v, dego_sp, degi_sp,
                 sa0, sb0, sa1, sb1):
    c = lax.axis_index("c")
    s = lax.axis_index("s")
    wid = c * NS + s
    iota = lax.iota(_i32, L)

    pltpu.sync_copy(i0_hbm.at[wid], i0_v)
    pltpu.sync_copy(i1_hbm.at[wid], i1_v)
    pltpu.async_copy(embn_hbm.at[i0_v.at[0]], a0_v, sa0)
    pltpu.async_copy(embn_hbm.at[i1_v.at[0]], b0_v, sb0)

    pltpu.sync_copy(srcn_hbm.at[wid], srcn_v)
    pltpu.sync_copy(dstn_hbm.at[wid], dstn_v)
    pltpu.sync_copy(wb_hbm.at[wid], wnb_v)

    def _z(i, _):
        zz_v[pl.ds(i * L, L)] = jnp.zeros((L,), _f32)
        return 0
    lax.fori_loop(0, STRIPE // L, _z, 0)
    pltpu.sync_copy(zz_v, dego_sp.at[pl.ds(s * STRIPE, STRIPE)])
    pltpu.sync_copy(zz_v, degi_sp.at[pl.ds(s * STRIPE, STRIPE)])
    plsc.subcore_barrier()

    bufs = ((a0_v, b0_v, sa0, sb0), (a1_v, b1_v, sa1, sb1))

    def _pred_pair(t, _):
        for par in range(2):
            ci = t * 2 + par
            abuf, bbuf, sa, sb = bufs[par]
            oabuf, obbuf, osa, osb = bufs[1 - par]
            pltpu.make_async_copy(embn_hbm.at[i0_v.at[ci]], abuf, sa).wait()
            pltpu.make_async_copy(embn_hbm.at[i1_v.at[ci]], bbuf, sb).wait()

            @pl.when(ci + 1 < NCH_P)
            def _():
                pltpu.async_copy(embn_hbm.at[i0_v.at[ci + 1]], oabuf, osa)
                pltpu.async_copy(embn_hbm.at[i1_v.at[ci + 1]], obbuf, osb)

            for q in range(2):
                nj = ci * 2 + q
                pltpu.sync_copy(wnb_v.at[nj], dego_sp.at[srcn_v.at[nj]], add=True)
                pltpu.sync_copy(wnb_v.at[nj], degi_sp.at[dstn_v.at[nj]], add=True)

            base = wid * PRED_PT + ci * CH

            def _group(g, _):
                def _edge(j, wvec):
                    e = g * L + j
                    acc = jnp.zeros((L,), _f32)
                    for k in range(EMB // L):
                        acc = acc + abuf[e, pl.ds(k * L, L)] * bbuf[e, pl.ds(k * L, L)]
                    cos = jnp.sum(acc)
                    valid = (base + e) < NPRED
                    w_sc = jnp.where(valid & (cos >= TAU), cos + EOS, 0.0)
                    return jnp.where(iota == j, w_sc, wvec)
                w16 = lax.fori_loop(0, L, _edge, jnp.zeros((L,), _f32))
                wst_v[ci, pl.ds(g * L, L)] = w16
                return 0
            lax.fori_loop(0, CH // L, _group, 0)

            pltpu.sync_copy(wst_v.at[ci], dego_sp.at[i0_v.at[ci]], add=True)
            pltpu.sync_copy(wst_v.at[ci], degi_sp.at[i1_v.at[ci]], add=True)
        return 0
    lax.fori_loop(0, NCH_P // 2, _pred_pair, 0)
    pltpu.sync_copy(wst_v, wpred_hbm.at[wid])

    def _non_chunk(ci, _):
        pltpu.sync_copy(wnb_v.at[ci], dego_sp.at[srcn_v.at[ci]], add=True)
        pltpu.sync_copy(wnb_v.at[ci], degi_sp.at[dstn_v.at[ci]], add=True)
        return 0
    lax.fori_loop(2 * NCH_P, NCH_N, _non_chunk, 0)

    plsc.subcore_barrier()
    pltpu.sync_copy(dego_sp.at[pl.ds(s * STRIPE, STRIPE)],
                    deg_hbm.at[c, 0, pl.ds(s * STRIPE, STRIPE)])
    pltpu.sync_copy(degi_sp.at[pl.ds(s * STRIPE, STRIPE)],
                    deg_hbm.at[c, 1, pl.ds(s * STRIPE, STRIPE)])


def _prop_body(h_hbm, src_hbm, dst_hbm, w_hbm,
               p_hbm,
               sb_v, db_v, wb_v, rows0, rows1, acc_sp, g0, g1, s0, s1):
    c = lax.axis_index("c")
    s = lax.axis_index("s")
    wid = c * NS + s

    pltpu.sync_copy(src_hbm.at[wid], sb_v)
    pltpu.async_copy(h_hbm.at[sb_v.at[0]], rows1, g1)

    def _zr(r, _):
        for k in range(EMB // L):
            rows0[r, pl.ds(k * L, L)] = jnp.zeros((L,), _f32)
        return 0
    lax.fori_loop(0, CH, _zr, 0)
    for j in range(STRIPE // CH):
        pltpu.async_copy(rows0, acc_sp.at[pl.ds(s * STRIPE + j * CH, CH)], g0)
    for j in range(STRIPE // CH):
        pltpu.make_async_copy(rows0, acc_sp.at[pl.ds(s * STRIPE, CH)], g0).wait()
    plsc.subcore_barrier()

    bufs = ((rows1, g1, s1), (rows0, g0, s0))

    def _block(bi, _):
        pltpu.sync_copy(dst_hbm.at[wid, pl.ds(bi * BLK, BLK)], db_v)
        pltpu.sync_copy(w_hbm.at[wid, pl.ds(bi * BLK, BLK)], wb_v)

        def _pair(t, _):
            for par in range(2):
                lc = t * 2 + par
                ci = bi * BLK + lc
                buf, gsem, ssem = bufs[par]
                obuf, ogsem, ossem = bufs[1 - par]

                pltpu.make_async_copy(h_hbm.at[sb_v.at[ci]], buf, gsem).wait()

                @pl.when(ci > 0)
                def _():
                    pltpu.make_async_copy(obuf, acc_sp.at[db_v.at[lc]], ossem).wait()

                @pl.when(ci + 1 < NCH_A)
                def _():
                    pltpu.async_copy(h_hbm.at[sb_v.at[ci + 1]], obuf, ogsem)

                def _sg(g, _):
                    w16 = wb_v[lc, pl.ds(g * L, L)]
                    for j in range(L):
                        b = w16[j]
                        e = g * L + j
                        for k in range(EMB // L):
                            buf[e, pl.ds(k * L, L)] = buf[e, pl.ds(k * L, L)] * b
                    return 0
                lax.fori_loop(0, CH // L, _sg, 0)

                pltpu.async_copy(buf, acc_sp.at[db_v.at[lc]], ssem, add=True)
            return 0
        lax.fori_loop(0, BLK // 2, _pair, 0)
        return 0
    lax.fori_loop(0, NCH_A // BLK, _block, 0)

    pltpu.make_async_copy(rows0, acc_sp.at[db_v.at[BLK - 1]], s0).wait()

    plsc.subcore_barrier()
    for j in range(STRIPE // CH):
        sl = pl.ds(s * STRIPE + j * CH, CH)
        pltpu.async_copy(acc_sp.at[sl], p_hbm.at[c, sl], g0)
    for j in range(STRIPE // CH):
        pltpu.make_async_copy(acc_sp.at[pl.ds(s * STRIPE, CH)],
                              p_hbm.at[c, pl.ds(s * STRIPE, CH)], g0).wait()


def kernel(features, embedding, weights_lp, W_lin, b_lin, W_mlp, b_mlp,
           edges, pred_edge_index):
    edges = edges.astype(_i32)
    pei = pred_edge_index.astype(_i32)

    npp = PPAD - NPRED
    i0p = jnp.concatenate([pei[0], jnp.arange(npp, dtype=_i32) % NNODES])
    i1p = jnp.concatenate([pei[1], (jnp.arange(npp, dtype=_i32) + 7) % NNODES])
    nnp = NONPAD - (NEDGES + NNODES)
    sl = jnp.arange(NNODES, dtype=_i32)
    srcn = jnp.concatenate([edges[0], sl, jnp.arange(nnp, dtype=_i32) % NNODES])
    dstn = jnp.concatenate([edges[1], sl, (jnp.arange(nnp, dtype=_i32) + 13) % NNODES])
    w_base = jnp.concatenate([
        weights_lp + EOS,
        jnp.ones((NNODES,), _f32) + EOS,
        jnp.zeros((nnp,), _f32),
    ])
    i0p3 = i0p.reshape(NW, NCH_P, CH)
    i1p3 = i1p.reshape(NW, NCH_P, CH)
    srcn3 = srcn.reshape(NW, NCH_N, CH)
    dstn3 = dstn.reshape(NW, NCH_N, CH)
    wb3 = w_base.reshape(NW, NCH_N, CH)

    emb_n = pl.pallas_call(
        _emb_body,
        out_shape=jax.ShapeDtypeStruct((NNODES, EMB), _f32),
    )(embedding)
    h = pl.pallas_call(
        _h_body,
        out_shape=jax.ShapeDtypeStruct((NNODES, EMB), _f32),
    )(features, W_lin.T, b_lin.reshape(1, EMB))

    sc_mesh = plsc.VectorSubcoreMesh(core_axis_name="c", subcore_axis_name="s")
    edge_w = pl.kernel(
        _edge_w_body,
        out_type=[jax.ShapeDtypeStruct((NW, NCH_P, CH), _f32),
                  jax.ShapeDtypeStruct((NC, 2, NPADN), _f32)],
        mesh=sc_mesh,
        compiler_params=_SC_PARAMS,
        scratch_types=[
            pltpu.VMEM((NCH_P, CH), _i32),
            pltpu.VMEM((NCH_P, CH), _i32),
            pltpu.VMEM((CH, EMB), _f32),
            pltpu.VMEM((CH, EMB), _f32),
            pltpu.VMEM((CH, EMB), _f32),
            pltpu.VMEM((CH, EMB), _f32),
            pltpu.VMEM((NCH_P, CH), _f32),
            pltpu.VMEM((NCH_N, CH), _f32),
            pltpu.VMEM((NCH_N, CH), _i32),
            pltpu.VMEM((NCH_N, CH), _i32),
            pltpu.VMEM((STRIPE,), _f32),
            pltpu.VMEM_SHARED((NPADN,), _f32),
            pltpu.VMEM_SHARED((NPADN,), _f32),
            pltpu.SemaphoreType.DMA,
            pltpu.SemaphoreType.DMA,
            pltpu.SemaphoreType.DMA,
            pltpu.SemaphoreType.DMA,
        ],
    )
    wpred3, deg = edge_w(emb_n, i0p3, i1p3, srcn3, dstn3, wb3)

    deg_t = deg.reshape(NC * 2, NPADN).T
    h2, rsi_col = pl.pallas_call(
        _rs_body,
        out_shape=[jax.ShapeDtypeStruct((NNODES, EMB), _f32),
                   jax.ShapeDtypeStruct((NPADN, 1), _f32)],
    )(deg_t, h)

    src_all = jnp.concatenate([i0p, srcn]).reshape(NW, NCH_A, CH)
    dst_all = jnp.concatenate([i1p, dstn]).reshape(NW, NCH_A, CH)
    w_all = jnp.concatenate([wpred3.reshape(PPAD), w_base]).reshape(NW, NCH_A, CH)

    prop = pl.kernel(
        _prop_body,
        out_type=jax.ShapeDtypeStruct((NC, NPADN, EMB), _f32),
        mesh=sc_mesh,
        compiler_params=_SC_PARAMS,
        scratch_types=[
            pltpu.VMEM((NCH_A, CH), _i32),
            pltpu.VMEM((BLK, CH), _i32),
            pltpu.VMEM((BLK, CH), _f32),
            pltpu.VMEM((CH, EMB), _f32),
            pltpu.VMEM((CH, EMB), _f32),
            pltpu.VMEM_SHARED((NPADN, EMB), _f32),
            pltpu.SemaphoreType.DMA,
            pltpu.SemaphoreType.DMA,
            pltpu.SemaphoreType.DMA,
            pltpu.SemaphoreType.DMA,
        ],
    )
    p = prop(h2, src_all, dst_all, w_all)

    wm_pad = jnp.zeros((EMB, EMB), _f32).at[:, :NCLASS].set(W_mlp.T)
    bm_pad = jnp.zeros((1, EMB), _f32).at[0, :NCLASS].set(b_mlp)
    out_pad = pl.pallas_call(
        _mlp_body,
        out_shape=jax.ShapeDtypeStruct((NPADN, EMB), _f32),
    )(p, rsi_col, wm_pad, bm_pad)
    return out_pad[:NNODES, :NCLASS]


# speedup vs baseline: 1.1153x; 1.0036x over previous
import functools

import jax
import jax.numpy as jnp
from jax import lax
from jax.experimental import pallas as pl
from jax.experimental.pallas import tpu as pltpu
from jax.experimental.pallas import tpu_sc as plsc

EOS = 1e-10
NNODES = 10000
NEDGES = 160000
DIN = 256
EMB = 128
NCLASS = 16
NPRED = 80000
TAU = 0.1

NC = 2
NS = 16
NW = NC * NS
L = 16

NPADN = 10240
STRIPE = NPADN // NS

CH = 128
NCH_P = 20
PRED_PT = NCH_P * CH
PPAD = NW * PRED_PT
NCH_N = 44
NON_PT = NCH_N * CH
NONPAD = NW * NON_PT
NCH_A = NCH_P + NCH_N
BLK = 8

_f32 = jnp.float32
_i32 = jnp.int32

_SC_PARAMS = pltpu.CompilerParams(needs_layout_passes=False)


def _h_body(x_ref, wl_ref, bl_ref, h_ref):
    x = x_ref[...]
    h = jnp.dot(x, wl_ref[...], preferred_element_type=_f32) + bl_ref[...]
    h_ref[...] = jnp.maximum(h, 0.0)


def _emb_body(emb_ref, embn_ref):
    e = emb_ref[...]
    n2 = jnp.sum(e * e, axis=1, keepdims=True)
    embn_ref[...] = e * lax.rsqrt(jnp.maximum(n2, 1e-16))


def _rs_body(degt_ref, h_ref, h2_ref, rsi_ref):
    d = degt_ref[...]
    tot_o = d[:, 0:1] + d[:, 2:3]
    tot_i = d[:, 1:2] + d[:, 3:4]
    rso = lax.rsqrt(jnp.maximum(tot_o, 1e-12))
    rsi_ref[...] = lax.rsqrt(jnp.maximum(tot_i, 1e-12))
    h2_ref[...] = h_ref[...] * rso[:NNODES]


def _mlp_body(p_ref, rsi_ref, wm_ref, bm_ref, o_ref):
    hsum = (p_ref[0] + p_ref[1]) * rsi_ref[...]
    o_ref[...] = jnp.dot(hsum, wm_ref[...], preferred_element_type=_f32) + bm_ref[...]


def _edge_w_body(embn_hbm, i0_hbm, i1_hbm, srcn_hbm, dstn_hbm, wb_hbm,
                 wpred_hbm, deg_hbm,
                 i0_v, i1_v, a0_v, b0_v, a1_v, b1_v, wst_v, wnb_v,
                 srcn_v, dstn_v, zz_v, dego_sp, degi_sp,
                 sa0, sb0, sa1, sb1):
    c = lax.axis_index("c")
    s = lax.axis_index("s")
    wid = c * NS + s
    iota = lax.iota(_i32, L)

    pltpu.sync_copy(i0_hbm.at[wid], i0_v)
    pltpu.sync_copy(i1_hbm.at[wid], i1_v)
    pltpu.async_copy(embn_hbm.at[i0_v.at[0]], a0_v, sa0)
    pltpu.async_copy(embn_hbm.at[i1_v.at[0]], b0_v, sb0)

    pltpu.sync_copy(srcn_hbm.at[wid], srcn_v)
    pltpu.sync_copy(dstn_hbm.at[wid], dstn_v)
    pltpu.sync_copy(wb_hbm.at[wid], wnb_v)

    def _z(i, _):
        zz_v[pl.ds(i * L, L)] = jnp.zeros((L,), _f32)
        return 0
    lax.fori_loop(0, STRIPE // L, _z, 0)
    pltpu.sync_copy(zz_v, dego_sp.at[pl.ds(s * STRIPE, STRIPE)])
    pltpu.sync_copy(zz_v, degi_sp.at[pl.ds(s * STRIPE, STRIPE)])
    plsc.subcore_barrier()

    bufs = ((a0_v, b0_v, sa0, sb0), (a1_v, b1_v, sa1, sb1))

    def _pred_pair(t, _):
        for par in range(2):
            ci = t * 2 + par
            abuf, bbuf, sa, sb = bufs[par]
            oabuf, obbuf, osa, osb = bufs[1 - par]
            pltpu.make_async_copy(embn_hbm.at[i0_v.at[ci]], abuf, sa).wait()
            pltpu.make_async_copy(embn_hbm.at[i1_v.at[ci]], bbuf, sb).wait()

            @pl.when(ci + 1 < NCH_P)
            def _():
                pltpu.async_copy(embn_hbm.at[i0_v.at[ci + 1]], oabuf, osa)
                pltpu.async_copy(embn_hbm.at[i1_v.at[ci + 1]], obbuf, osb)

            for q in range(2):
                nj = ci * 2 + q
                pltpu.sync_copy(wnb_v.at[nj], dego_sp.at[srcn_v.at[nj]], add=True)
                pltpu.sync_copy(wnb_v.at[nj], degi_sp.at[dstn_v.at[nj]], add=True)

            base = wid * PRED_PT + ci * CH

            def _group(g, _):
                def _edge(j, wvec):
                    e = g * L + j
                    acc = jnp.zeros((L,), _f32)
                    for k in range(EMB // L):
                        acc = acc + abuf[e, pl.ds(k * L, L)] * bbuf[e, pl.ds(k * L, L)]
                    cos = jnp.sum(acc)
                    valid = (base + e) < NPRED
                    w_sc = jnp.where(valid & (cos >= TAU), cos + EOS, 0.0)
                    return jnp.where(iota == j, w_sc, wvec)
                w16 = lax.fori_loop(0, L, _edge, jnp.zeros((L,), _f32))
                wst_v[ci, pl.ds(g * L, L)] = w16
                return 0
            lax.fori_loop(0, CH // L, _group, 0)

            pltpu.sync_copy(wst_v.at[ci], dego_sp.at[i0_v.at[ci]], add=True)
            pltpu.sync_copy(wst_v.at[ci], degi_sp.at[i1_v.at[ci]], add=True)
        return 0
    lax.fori_loop(0, NCH_P // 2, _pred_pair, 0)
    pltpu.sync_copy(wst_v, wpred_hbm.at[wid])

    def _non_chunk(ci, _):
        pltpu.sync_copy(wnb_v.at[ci], dego_sp.at[srcn_v.at[ci]], add=True)
        pltpu.sync_copy(wnb_v.at[ci], degi_sp.at[dstn_v.at[ci]], add=True)
        return 0
    lax.fori_loop(2 * NCH_P, NCH_N, _non_chunk, 0)

    plsc.subcore_barrier()
    pltpu.sync_copy(dego_sp.at[pl.ds(s * STRIPE, STRIPE)],
                    deg_hbm.at[c, 0, pl.ds(s * STRIPE, STRIPE)])
    pltpu.sync_copy(degi_sp.at[pl.ds(s * STRIPE, STRIPE)],
                    deg_hbm.at[c, 1, pl.ds(s * STRIPE, STRIPE)])


def _prop_body(h_hbm, src_hbm, dst_hbm, w_hbm,
               p_hbm,
               sb_v, db_v, wb_v, rows0, rows1, acc_sp, g0, g1, s0, s1):
    c = lax.axis_index("c")
    s = lax.axis_index("s")
    wid = c * NS + s

    pltpu.sync_copy(src_hbm.at[wid], sb_v)
    pltpu.async_copy(h_hbm.at[sb_v.at[0, pl.ds(0, CH // 2)]],
                     rows1.at[pl.ds(0, CH // 2)], g1)
    pltpu.async_copy(h_hbm.at[sb_v.at[0, pl.ds(CH // 2, CH // 2)]],
                     rows1.at[pl.ds(CH // 2, CH // 2)], g1)

    def _zr(r, _):
        for k in range(EMB // L):
            rows0[r, pl.ds(k * L, L)] = jnp.zeros((L,), _f32)
        return 0
    lax.fori_loop(0, CH, _zr, 0)
    for j in range(STRIPE // CH):
        pltpu.async_copy(rows0, acc_sp.at[pl.ds(s * STRIPE + j * CH, CH)], g0)
    for j in range(STRIPE // CH):
        pltpu.make_async_copy(rows0, acc_sp.at[pl.ds(s * STRIPE, CH)], g0).wait()
    plsc.subcore_barrier()

    bufs = ((rows1, g1, s1), (rows0, g0, s0))

    def _block(bi, _):
        pltpu.sync_copy(dst_hbm.at[wid, pl.ds(bi * BLK, BLK)], db_v)
        pltpu.sync_copy(w_hbm.at[wid, pl.ds(bi * BLK, BLK)], wb_v)

        def _pair(t, _):
            for par in range(2):
                lc = t * 2 + par
                ci = bi * BLK + lc
                buf, gsem, ssem = bufs[par]
                obuf, ogsem, ossem = bufs[1 - par]

                pltpu.make_async_copy(h_hbm.at[sb_v.at[ci, pl.ds(0, CH // 2)]],
                                      buf.at[pl.ds(0, CH // 2)], gsem).wait()
                pltpu.make_async_copy(h_hbm.at[sb_v.at[ci, pl.ds(CH // 2, CH // 2)]],
                                      buf.at[pl.ds(CH // 2, CH // 2)], gsem).wait()

                @pl.when(ci > 0)
                def _():
                    pltpu.make_async_copy(obuf, acc_sp.at[db_v.at[lc]], ossem).wait()

                @pl.when(ci + 1 < NCH_A)
                def _():
                    pltpu.async_copy(h_hbm.at[sb_v.at[ci + 1, pl.ds(0, CH // 2)]],
                                     obuf.at[pl.ds(0, CH // 2)], ogsem)
                    pltpu.async_copy(h_hbm.at[sb_v.at[ci + 1, pl.ds(CH // 2, CH // 2)]],
                                     obuf.at[pl.ds(CH // 2, CH // 2)], ogsem)

                def _sg(g, _):
                    w16 = wb_v[lc, pl.ds(g * L, L)]
                    for j in range(L):
                        b = w16[j]
                        e = g * L + j
                        for k in range(EMB // L):
                            buf[e, pl.ds(k * L, L)] = buf[e, pl.ds(k * L, L)] * b
                    return 0
                lax.fori_loop(0, CH // L, _sg, 0)

                pltpu.async_copy(buf, acc_sp.at[db_v.at[lc]], ssem, add=True)
            return 0
        lax.fori_loop(0, BLK // 2, _pair, 0)
        return 0
    lax.fori_loop(0, NCH_A // BLK, _block, 0)

    pltpu.make_async_copy(rows0, acc_sp.at[db_v.at[BLK - 1]], s0).wait()

    plsc.subcore_barrier()
    for j in range(STRIPE // CH):
        sl = pl.ds(s * STRIPE + j * CH, CH)
        pltpu.async_copy(acc_sp.at[sl], p_hbm.at[c, sl], g0)
    for j in range(STRIPE // CH):
        pltpu.make_async_copy(acc_sp.at[pl.ds(s * STRIPE, CH)],
                              p_hbm.at[c, pl.ds(s * STRIPE, CH)], g0).wait()


def kernel(features, embedding, weights_lp, W_lin, b_lin, W_mlp, b_mlp,
           edges, pred_edge_index):
    edges = edges.astype(_i32)
    pei = pred_edge_index.astype(_i32)

    npp = PPAD - NPRED
    i0p = jnp.concatenate([pei[0], jnp.arange(npp, dtype=_i32) % NNODES])
    i1p = jnp.concatenate([pei[1], (jnp.arange(npp, dtype=_i32) + 7) % NNODES])
    nnp = NONPAD - (NEDGES + NNODES)
    sl = jnp.arange(NNODES, dtype=_i32)
    srcn = jnp.concatenate([edges[0], sl, jnp.arange(nnp, dtype=_i32) % NNODES])
    dstn = jnp.concatenate([edges[1], sl, (jnp.arange(nnp, dtype=_i32) + 13) % NNODES])
    w_base = jnp.concatenate([
        weights_lp + EOS,
        jnp.ones((NNODES,), _f32) + EOS,
        jnp.zeros((nnp,), _f32),
    ])
    i0p3 = i0p.reshape(NW, NCH_P, CH)
    i1p3 = i1p.reshape(NW, NCH_P, CH)
    srcn3 = srcn.reshape(NW, NCH_N, CH)
    dstn3 = dstn.reshape(NW, NCH_N, CH)
    wb3 = w_base.reshape(NW, NCH_N, CH)

    emb_n = pl.pallas_call(
        _emb_body,
        out_shape=jax.ShapeDtypeStruct((NNODES, EMB), _f32),
    )(embedding)
    h = pl.pallas_call(
        _h_body,
        out_shape=jax.ShapeDtypeStruct((NNODES, EMB), _f32),
    )(features, W_lin.T, b_lin.reshape(1, EMB))

    sc_mesh = plsc.VectorSubcoreMesh(core_axis_name="c", subcore_axis_name="s")
    edge_w = pl.kernel(
        _edge_w_body,
        out_type=[jax.ShapeDtypeStruct((NW, NCH_P, CH), _f32),
                  jax.ShapeDtypeStruct((NC, 2, NPADN), _f32)],
        mesh=sc_mesh,
        compiler_params=_SC_PARAMS,
        scratch_types=[
            pltpu.VMEM((NCH_P, CH), _i32),
            pltpu.VMEM((NCH_P, CH), _i32),
            pltpu.VMEM((CH, EMB), _f32),
            pltpu.VMEM((CH, EMB), _f32),
            pltpu.VMEM((CH, EMB), _f32),
            pltpu.VMEM((CH, EMB), _f32),
            pltpu.VMEM((NCH_P, CH), _f32),
            pltpu.VMEM((NCH_N, CH), _f32),
            pltpu.VMEM((NCH_N, CH), _i32),
            pltpu.VMEM((NCH_N, CH), _i32),
            pltpu.VMEM((STRIPE,), _f32),
            pltpu.VMEM_SHARED((NPADN,), _f32),
            pltpu.VMEM_SHARED((NPADN,), _f32),
            pltpu.SemaphoreType.DMA,
            pltpu.SemaphoreType.DMA,
            pltpu.SemaphoreType.DMA,
            pltpu.SemaphoreType.DMA,
        ],
    )
    wpred3, deg = edge_w(emb_n, i0p3, i1p3, srcn3, dstn3, wb3)

    deg_t = deg.reshape(NC * 2, NPADN).T
    h2, rsi_col = pl.pallas_call(
        _rs_body,
        out_shape=[jax.ShapeDtypeStruct((NNODES, EMB), _f32),
                   jax.ShapeDtypeStruct((NPADN, 1), _f32)],
    )(deg_t, h)

    src_all = jnp.concatenate([i0p, srcn]).reshape(NW, NCH_A, CH)
    dst_all = jnp.concatenate([i1p, dstn]).reshape(NW, NCH_A, CH)
    w_all = jnp.concatenate([wpred3.reshape(PPAD), w_base]).reshape(NW, NCH_A, CH)

    prop = pl.kernel(
        _prop_body,
        out_type=jax.ShapeDtypeStruct((NC, NPADN, EMB), _f32),
        mesh=sc_mesh,
        compiler_params=_SC_PARAMS,
        scratch_types=[
            pltpu.VMEM((NCH_A, CH), _i32),
            pltpu.VMEM((BLK, CH), _i32),
            pltpu.VMEM((BLK, CH), _f32),
            pltpu.VMEM((CH, EMB), _f32),
            pltpu.VMEM((CH, EMB), _f32),
            pltpu.VMEM_SHARED((NPADN, EMB), _f32),
            pltpu.SemaphoreType.DMA,
            pltpu.SemaphoreType.DMA,
            pltpu.SemaphoreType.DMA,
            pltpu.SemaphoreType.DMA,
        ],
    )
    p = prop(h2, src_all, dst_all, w_all)

    wm_pad = jnp.zeros((EMB, EMB), _f32).at[:, :NCLASS].set(W_mlp.T)
    bm_pad = jnp.zeros((1, EMB), _f32).at[0, :NCLASS].set(b_mlp)
    out_pad = pl.pallas_call(
        _mlp_body,
        out_shape=jax.ShapeDtypeStruct((NPADN, EMB), _f32),
    )(p, rsi_col, wm_pad, bm_pad)
    return out_pad[:NNODES, :NCLASS]
